# baseline (speedup 1.0000x reference)
import jax
import jax.numpy as jnp
from jax import lax
from jax.experimental import pallas as pl
from jax.experimental.pallas import tpu as pltpu
from jax.experimental.pallas import tpu_sc as plsc

B = 16384
D = 64
V = 65536
NC = 2
NS = 16
NW = NC * NS
CPT = B // NS
CHUNK = 128
FPT = D // NW


def _body(xt_hbm, ind_hbm, wt_hbm, map_hbm, out_hbm,
          idx_v, rows_v, rows_full, wrow_v, acc_v, rows_sh,
          sem_i, sem_m, sem_w, sem_x, sem_o):
    c = lax.axis_index("c")
    s = lax.axis_index("s")
    wid = c * NS + s

    d0 = wid * FPT
    cp_w = pltpu.async_copy(wt_hbm.at[pl.ds(d0, 1)], wrow_v, sem_w)
    cp_x = pltpu.async_copy(xt_hbm.at[pl.ds(d0, 1)], acc_v.at[pl.ds(0, 1)],
                            sem_x)

    cbase = pl.multiple_of(s * CPT, CPT)
    pltpu.async_copy(ind_hbm.at[pl.ds(cbase, CPT)], idx_v, sem_i).wait()
    mcps = [
        pltpu.async_copy(
            map_hbm.at[idx_v.at[pl.ds(j * CHUNK, CHUNK)]],
            rows_v.at[pl.ds(j * CHUNK, CHUNK)],
            sem_m,
        )
        for j in range(CPT // CHUNK)
    ]
    for cp in mcps:
        cp.wait()
    pltpu.sync_copy(rows_v, rows_sh.at[pl.ds(cbase, CPT)])
    plsc.subcore_barrier()
    cp_r = pltpu.async_copy(rows_sh, rows_full, sem_i)

    cp_r.wait()
    for f in range(FPT):
        d = wid * FPT + f
        cp_w.wait()
        cp_x.wait()

        def vec_body(i):
            sl = pl.ds(i * 16, 16)
            idx = rows_full[sl]
            g = plsc.load_gather(wrow_v.at[0], [idx])
            plsc.addupdate(acc_v.at[f, sl], g)
        plsc.parallel_loop(0, B // 16, 1, unroll=16)(vec_body)

        if f + 1 < FPT:
            cp_w = pltpu.async_copy(
                wt_hbm.at[pl.ds(d + 1, 1)], wrow_v, sem_w)
            cp_x = pltpu.async_copy(
                xt_hbm.at[pl.ds(d + 1, 1)], acc_v.at[pl.ds(f + 1, 1)], sem_x)
        pltpu.async_copy(
            acc_v.at[pl.ds(f, 1)], out_hbm.at[pl.ds(d, 1)], sem_o)
    pltpu.make_async_copy(acc_v.at[pl.ds(0, 1)],
                          out_hbm.at[pl.ds(0, 1)], sem_o).wait()
    pltpu.make_async_copy(acc_v.at[pl.ds(0, 1)],
                          out_hbm.at[pl.ds(0, 1)], sem_o).wait()


@jax.jit
def kernel(x, ind, weight, mapping):
    ind32 = ind.astype(jnp.int32)
    map32 = mapping.astype(jnp.int32)
    xt = x.T
    wt = weight.T
    mesh = plsc.VectorSubcoreMesh(core_axis_name="c", subcore_axis_name="s")
    run = pl.kernel(
        _body,
        out_type=jax.ShapeDtypeStruct((D, B), jnp.float32),
        mesh=mesh,
        compiler_params=pltpu.CompilerParams(needs_layout_passes=False),
        scratch_types=[
            pltpu.VMEM((CPT,), jnp.int32),
            pltpu.VMEM((CPT,), jnp.int32),
            pltpu.VMEM((B,), jnp.int32),
            pltpu.VMEM((1, V), jnp.float32),
            pltpu.VMEM((FPT, B), jnp.float32),
            pltpu.VMEM_SHARED((B,), jnp.int32),
            pltpu.SemaphoreType.DMA,
            pltpu.SemaphoreType.DMA,
            pltpu.SemaphoreType.DMA,
            pltpu.SemaphoreType.DMA,
            pltpu.SemaphoreType.DMA,
        ],
    )
    return run(xt, ind32, wt, map32).T


# speedup vs baseline: 1.0003x; 1.0003x over previous
import jax
import jax.numpy as jnp
from jax import lax
from jax.experimental import pallas as pl
from jax.experimental.pallas import tpu as pltpu
from jax.experimental.pallas import tpu_sc as plsc

B = 16384
D = 64
V = 65536
NC = 2
NS = 16
NW = NC * NS
CPT = B // NS
CHUNK = 128
FPT = D // NW


def _body(xt_hbm, ind_hbm, wt_hbm, map_hbm, out_hbm,
          idx_v, rows_v, rows_full, wrow_v, acc_v, rows_sh,
          sem_i, sem_m, sem_w, sem_x, sem_o):
    c = lax.axis_index("c")
    s = lax.axis_index("s")
    wid = c * NS + s

    d0 = wid * FPT
    cp_w = pltpu.async_copy(wt_hbm.at[pl.ds(d0, 1)], wrow_v, sem_w)
    cp_x = pltpu.async_copy(xt_hbm.at[pl.ds(d0, 1)], acc_v.at[pl.ds(0, 1)],
                            sem_x)

    with jax.named_scope("p1_rows"):
        cbase = pl.multiple_of(s * CPT, CPT)
        pltpu.async_copy(ind_hbm.at[pl.ds(cbase, CPT)], idx_v, sem_i).wait()
        mcps = [
            pltpu.async_copy(
                map_hbm.at[idx_v.at[pl.ds(j * CHUNK, CHUNK)]],
                rows_v.at[pl.ds(j * CHUNK, CHUNK)],
                sem_m,
            )
            for j in range(CPT // CHUNK)
        ]
        for cp in mcps:
            cp.wait()
        pltpu.sync_copy(rows_v, rows_sh.at[pl.ds(cbase, CPT)])
        plsc.subcore_barrier()
    with jax.named_scope("p1_rowsfull"):
        pltpu.async_copy(rows_sh, rows_full, sem_i).wait()

    for f in range(FPT):
        d = wid * FPT + f
        with jax.named_scope(f"wait_wx{f}"):
            cp_w.wait()
            cp_x.wait()

        def vec_body(i):
            sl = pl.ds(i * 16, 16)
            idx = rows_full[sl]
            g = plsc.load_gather(wrow_v.at[0], [idx])
            plsc.addupdate(acc_v.at[f, sl], g)
        with jax.named_scope(f"compute{f}"):
            plsc.parallel_loop(0, B // 16, 1, unroll=16)(vec_body)

        if f + 1 < FPT:
            cp_w = pltpu.async_copy(
                wt_hbm.at[pl.ds(d + 1, 1)], wrow_v, sem_w)
            cp_x = pltpu.async_copy(
                xt_hbm.at[pl.ds(d + 1, 1)], acc_v.at[pl.ds(f + 1, 1)], sem_x)
        pltpu.async_copy(
            acc_v.at[pl.ds(f, 1)], out_hbm.at[pl.ds(d, 1)], sem_o)
    pltpu.make_async_copy(acc_v.at[pl.ds(0, 1)],
                          out_hbm.at[pl.ds(0, 1)], sem_o).wait()
    pltpu.make_async_copy(acc_v.at[pl.ds(0, 1)],
                          out_hbm.at[pl.ds(0, 1)], sem_o).wait()


@jax.jit
def kernel(x, ind, weight, mapping):
    ind32 = ind.astype(jnp.int32)
    map32 = mapping.astype(jnp.int32)
    xt = x.T
    wt = weight.T
    mesh = plsc.VectorSubcoreMesh(core_axis_name="c", subcore_axis_name="s")
    run = pl.kernel(
        _body,
        out_type=jax.ShapeDtypeStruct((D, B), jnp.float32),
        mesh=mesh,
        compiler_params=pltpu.CompilerParams(needs_layout_passes=False),
        scratch_types=[
            pltpu.VMEM((CPT,), jnp.int32),
            pltpu.VMEM((CPT,), jnp.int32),
            pltpu.VMEM((B,), jnp.int32),
            pltpu.VMEM((1, V), jnp.float32),
            pltpu.VMEM((FPT, B), jnp.float32),
            pltpu.VMEM_SHARED((B,), jnp.int32),
            pltpu.SemaphoreType.DMA,
            pltpu.SemaphoreType.DMA,
            pltpu.SemaphoreType.DMA,
            pltpu.SemaphoreType.DMA,
            pltpu.SemaphoreType.DMA,
        ],
    )
    return run(xt, ind32, wt, map32).T


# speedup vs baseline: 1.0041x; 1.0038x over previous
import jax
import jax.numpy as jnp
from jax import lax
from jax.experimental import pallas as pl
from jax.experimental.pallas import tpu as pltpu
from jax.experimental.pallas import tpu_sc as plsc

B = 16384
D = 64
V = 65536
NC = 2
NS = 16
NW = NC * NS
CPT = B // NS
CHUNK = 128
FPT = D // NW


def _body(xt_hbm, ind_hbm, wt_hbm, map_hbm, out_hbm,
          idx_v, rows_v, rows_full, wrow_v, acc_v, rows_sh,
          sem_i, sem_m, sem_w, sem_x, sem_o):
    c = lax.axis_index("c")
    s = lax.axis_index("s")
    wid = c * NS + s

    d0 = wid * FPT
    with jax.named_scope("p1_rows"):
        cbase = pl.multiple_of(s * CPT, CPT)
        pltpu.async_copy(ind_hbm.at[pl.ds(cbase, CPT)], idx_v, sem_i).wait()
        mcps = [
            pltpu.async_copy(
                map_hbm.at[idx_v.at[pl.ds(j * CHUNK, CHUNK)]],
                rows_v.at[pl.ds(j * CHUNK, CHUNK)],
                sem_m,
            )
            for j in range(CPT // CHUNK)
        ]
        cp_w = pltpu.async_copy(wt_hbm.at[pl.ds(d0, 1)], wrow_v, sem_w)
        cp_x = pltpu.async_copy(xt_hbm.at[pl.ds(d0, 1)],
                                acc_v.at[pl.ds(0, 1)], sem_x)
        for cp in mcps:
            cp.wait()
        pltpu.sync_copy(rows_v, rows_sh.at[pl.ds(cbase, CPT)])
        plsc.subcore_barrier()
    with jax.named_scope("p1_rowsfull"):
        pltpu.async_copy(rows_sh, rows_full, sem_i).wait()

    for f in range(FPT):
        d = wid * FPT + f
        with jax.named_scope(f"wait_wx{f}"):
            cp_w.wait()
            cp_x.wait()

        def vec_body(i):
            sl = pl.ds(i * 16, 16)
            idx = rows_full[sl]
            g = plsc.load_gather(wrow_v.at[0], [idx])
            plsc.addupdate(acc_v.at[f, sl], g)
        with jax.named_scope(f"compute{f}"):
            plsc.parallel_loop(0, B // 16, 1, unroll=16)(vec_body)

        if f + 1 < FPT:
            cp_w = pltpu.async_copy(
                wt_hbm.at[pl.ds(d + 1, 1)], wrow_v, sem_w)
            cp_x = pltpu.async_copy(
                xt_hbm.at[pl.ds(d + 1, 1)], acc_v.at[pl.ds(f + 1, 1)], sem_x)
        pltpu.async_copy(
            acc_v.at[pl.ds(f, 1)], out_hbm.at[pl.ds(d, 1)], sem_o)
    pltpu.make_async_copy(acc_v.at[pl.ds(0, 1)],
                          out_hbm.at[pl.ds(0, 1)], sem_o).wait()
    pltpu.make_async_copy(acc_v.at[pl.ds(0, 1)],
                          out_hbm.at[pl.ds(0, 1)], sem_o).wait()


@jax.jit
def kernel(x, ind, weight, mapping):
    ind32 = ind.astype(jnp.int32)
    map32 = mapping.astype(jnp.int32)
    xt = x.T
    wt = weight.T
    mesh = plsc.VectorSubcoreMesh(core_axis_name="c", subcore_axis_name="s")
    run = pl.kernel(
        _body,
        out_type=jax.ShapeDtypeStruct((D, B), jnp.float32),
        mesh=mesh,
        compiler_params=pltpu.CompilerParams(needs_layout_passes=False),
        scratch_types=[
            pltpu.VMEM((CPT,), jnp.int32),
            pltpu.VMEM((CPT,), jnp.int32),
            pltpu.VMEM((B,), jnp.int32),
            pltpu.VMEM((1, V), jnp.float32),
            pltpu.VMEM((FPT, B), jnp.float32),
            pltpu.VMEM_SHARED((B,), jnp.int32),
            pltpu.SemaphoreType.DMA,
            pltpu.SemaphoreType.DMA,
            pltpu.SemaphoreType.DMA,
            pltpu.SemaphoreType.DMA,
            pltpu.SemaphoreType.DMA,
        ],
    )
    return run(xt, ind32, wt, map32).T


# speedup vs baseline: 1.0065x; 1.0024x over previous
import jax
import jax.numpy as jnp
from jax import lax
from jax.experimental import pallas as pl
from jax.experimental.pallas import tpu as pltpu
from jax.experimental.pallas import tpu_sc as plsc

B = 16384
D = 64
V = 65536
NC = 2
NS = 16
NW = NC * NS
CPT = B // NS
CHUNK = 128
FPT = D // NW


def _body(xt_hbm, ind_hbm, wt_hbm, map_hbm, out_hbm,
          idx_v, rows_v, rows_full, wrow_v, acc_v, rows_sh,
          sem_i, sem_m, sem_w, sem_x, sem_o):
    c = lax.axis_index("c")
    s = lax.axis_index("s")
    wid = c * NS + s

    d0 = wid * FPT
    cbase = pl.multiple_of(s * CPT, CPT)
    pltpu.async_copy(ind_hbm.at[pl.ds(cbase, CPT)], idx_v, sem_i).wait()
    mcps = [
        pltpu.async_copy(
            map_hbm.at[idx_v.at[pl.ds(j * CHUNK, CHUNK)]],
            rows_v.at[pl.ds(j * CHUNK, CHUNK)],
            sem_m,
        )
        for j in range(CPT // CHUNK)
    ]
    cp_w = pltpu.async_copy(wt_hbm.at[pl.ds(d0, 1)], wrow_v, sem_w)
    cp_x = pltpu.async_copy(xt_hbm.at[pl.ds(d0, 1)],
                            acc_v.at[pl.ds(0, 1)], sem_x)
    for cp in mcps:
        cp.wait()
    pltpu.sync_copy(rows_v, rows_sh.at[pl.ds(cbase, CPT)])
    plsc.subcore_barrier()
    pltpu.async_copy(rows_sh, rows_full, sem_i).wait()

    for f in range(FPT):
        d = wid * FPT + f
        cp_w.wait()
        cp_x.wait()

        def vec_body(i):
            sl = pl.ds(i * 16, 16)
            idx = rows_full[sl]
            g = plsc.load_gather(wrow_v.at[0], [idx])
            plsc.addupdate(acc_v.at[f, sl], g)
        plsc.parallel_loop(0, B // 16, 1, unroll=16)(vec_body)

        if f + 1 < FPT:
            cp_w = pltpu.async_copy(
                wt_hbm.at[pl.ds(d + 1, 1)], wrow_v, sem_w)
            cp_x = pltpu.async_copy(
                xt_hbm.at[pl.ds(d + 1, 1)], acc_v.at[pl.ds(f + 1, 1)], sem_x)
        pltpu.async_copy(
            acc_v.at[pl.ds(f, 1)], out_hbm.at[pl.ds(d, 1)], sem_o)
    pltpu.make_async_copy(acc_v.at[pl.ds(0, 1)],
                          out_hbm.at[pl.ds(0, 1)], sem_o).wait()
    pltpu.make_async_copy(acc_v.at[pl.ds(0, 1)],
                          out_hbm.at[pl.ds(0, 1)], sem_o).wait()


@jax.jit
def kernel(x, ind, weight, mapping):
    ind32 = ind.astype(jnp.int32)
    map32 = mapping.astype(jnp.int32)
    xt = x.T
    wt = weight.T
    mesh = plsc.VectorSubcoreMesh(core_axis_name="c", subcore_axis_name="s")
    run = pl.kernel(
        _body,
        out_type=jax.ShapeDtypeStruct((D, B), jnp.float32),
        mesh=mesh,
        compiler_params=pltpu.CompilerParams(needs_layout_passes=False),
        scratch_types=[
            pltpu.VMEM((CPT,), jnp.int32),
            pltpu.VMEM((CPT,), jnp.int32),
            pltpu.VMEM((B,), jnp.int32),
            pltpu.VMEM((1, V), jnp.float32),
            pltpu.VMEM((FPT, B), jnp.float32),
            pltpu.VMEM_SHARED((B,), jnp.int32),
            pltpu.SemaphoreType.DMA,
            pltpu.SemaphoreType.DMA,
            pltpu.SemaphoreType.DMA,
            pltpu.SemaphoreType.DMA,
            pltpu.SemaphoreType.DMA,
        ],
    )
    return run(xt, ind32, wt, map32).T
